# baseline (speedup 1.0000x reference)
import functools

import jax
import jax.numpy as jnp
from jax import lax
from jax.experimental import pallas as pl
from jax.experimental.pallas import tpu as pltpu
from jax.experimental.pallas import tpu_sc as plsc

NC = 2
NS = 16
NW = NC * NS
L = 16
D = 128
C = 64


@functools.partial(jax.jit, static_argnames=("e_pad",))
def _scores_padded(x, src2, dst2, e_pad):
    epw = e_pad // NW
    nchunk = epw // C
    groups = C // L
    n_nodes = x.shape[0]

    mesh = plsc.VectorSubcoreMesh(core_axis_name="c", subcore_axis_name="s")

    @functools.partial(
        pl.kernel,
        out_type=jax.ShapeDtypeStruct((e_pad,), jnp.float32),
        mesh=mesh,
        scratch_types=[
            [pltpu.VMEM((C,), jnp.int32) for _ in range(3)],
            [pltpu.VMEM((C,), jnp.int32) for _ in range(3)],
            [pltpu.VMEM((C, D), jnp.float32) for _ in range(2)],
            [pltpu.VMEM((C, D), jnp.float32) for _ in range(2)],
            pltpu.VMEM((epw,), jnp.float32),
            pltpu.VMEM((L * 17,), jnp.float32),
            pltpu.VMEM_SHARED((n_nodes, D), jnp.float32),
            [pltpu.SemaphoreType.DMA for _ in range(3)],
            [pltpu.SemaphoreType.DMA for _ in range(3)],
            [pltpu.SemaphoreType.DMA for _ in range(2)],
            [pltpu.SemaphoreType.DMA for _ in range(2)],
        ],
        compiler_params=pltpu.CompilerParams(needs_layout_passes=False),
    )
    def k(x_hbm, src_hbm, dst_hbm, out_hbm,
          ids_s, ids_d, rows_s, rows_d, sc, tp, x_sh,
          sem_is, sem_id, sem_a, sem_b):
        sid = lax.axis_index("s")
        wid = sid * NC + lax.axis_index("c")
        base = wid * epw

        stripe = ((n_nodes // NS) // 8) * 8
        tail = n_nodes - NS * stripe
        s_off = sid * stripe
        pltpu.sync_copy(x_hbm.at[pl.ds(s_off, stripe)],
                        x_sh.at[pl.ds(s_off, stripe)])
        if tail:
            @pl.when(sid == 0)
            def _():
                pltpu.sync_copy(x_hbm.at[pl.ds(NS * stripe, tail)],
                                x_sh.at[pl.ds(NS * stripe, tail)])
        plsc.subcore_barrier()

        def load_ids(chunk, q):
            off = base + chunk * C
            pltpu.async_copy(src_hbm.at[pl.ds(off, C)], ids_s[q], sem_is[q])
            pltpu.async_copy(dst_hbm.at[pl.ds(off, C)], ids_d[q], sem_id[q])

        def wait_ids(q):
            pltpu.make_async_copy(src_hbm.at[pl.ds(0, C)], ids_s[q], sem_is[q]).wait()
            pltpu.make_async_copy(dst_hbm.at[pl.ds(0, C)], ids_d[q], sem_id[q]).wait()

        def fetch(q, rp):
            pltpu.async_copy(x_sh.at[ids_s[q]], rows_s[rp], sem_a[rp])
            pltpu.async_copy(x_sh.at[ids_d[q]], rows_d[rp], sem_b[rp])

        def wait_rows(rp):
            pltpu.make_async_copy(x_hbm.at[pl.ds(0, C)], rows_s[rp], sem_a[rp]).wait()
            pltpu.make_async_copy(x_hbm.at[pl.ds(0, C)], rows_d[rp], sem_b[rp]).wait()

        def compute(chunk, rp):
            rs, rd = rows_s[rp], rows_d[rp]

            def group_body(g, carry2):
                for j in range(L):
                    e = g * L + j
                    ps = []
                    for kk in range(D // L):
                        a = rs[e, pl.ds(kk * L, L)]
                        b = rd[e, pl.ds(kk * L, L)]
                        ps.append(a * b)
                    s0 = (ps[0] + ps[1]) + (ps[2] + ps[3])
                    s1 = (ps[4] + ps[5]) + (ps[6] + ps[7])
                    tp[pl.ds(j * 17, L)] = s0 + s1
                base17 = lax.iota(jnp.int32, L) * 17
                acc0 = jnp.zeros((L,), jnp.float32)
                acc1 = jnp.zeros((L,), jnp.float32)
                for kk in range(L // 2):
                    acc0 = acc0 + plsc.load_gather(tp, [base17 + (2 * kk)])
                    acc1 = acc1 + plsc.load_gather(tp, [base17 + (2 * kk + 1)])
                sc[pl.ds(pl.multiple_of(chunk * C + g * L, L), L)] = acc0 + acc1
                return carry2

            lax.fori_loop(0, groups, group_body, 0, unroll=2)

        load_ids(jnp.int32(0), 0)
        load_ids(jnp.int32(1), 1)
        wait_ids(0)
        fetch(0, 0)

        def six_body(it, carry):
            c0 = 6 * it
            for b in range(6):
                c = c0 + b
                nxt = jnp.where(c + 1 == nchunk, 0, c + 1)
                nxt2 = jnp.where(c + 2 >= nchunk, c + 2 - nchunk, c + 2)
                wait_ids((b + 1) % 3)
                fetch((b + 1) % 3, (b + 1) % 2)
                load_ids(nxt2, (b + 2) % 3)
                wait_rows(b % 2)
                compute(c, b % 2)
            return carry

        lax.fori_loop(0, nchunk // 6, six_body, 0, unroll=False)
        wait_rows(0)
        wait_ids(1)
        pltpu.sync_copy(sc, out_hbm.at[pl.ds(base, epw)])

    return k(x, src2, dst2)


def kernel(x, edge_index):
    e = edge_index.shape[1]
    quantum = NW * C * 6
    e_pad = ((e + quantum - 1) // quantum) * quantum
    src = edge_index[0]
    dst = edge_index[1]
    if e_pad != e:
        pad = jnp.zeros((e_pad - e,), jnp.int32)
        src = jnp.concatenate([src, pad])
        dst = jnp.concatenate([dst, pad])
    out = _scores_padded(x, src, dst, e_pad)
    return out[:e]


# speedup vs baseline: 1.5110x; 1.5110x over previous
import functools

import jax
import jax.numpy as jnp
from jax import lax
from jax.experimental import pallas as pl
from jax.experimental.pallas import tpu as pltpu
from jax.experimental.pallas import tpu_sc as plsc

NC = 2
NS = 16
NW = NC * NS
L = 16
D = 128
C = 64


@functools.partial(jax.jit, static_argnames=("e_pad",))
def _scores_padded(x, src2, dst2, e_pad):
    epw = e_pad // NW
    nchunk = epw // C
    groups = C // L
    n_nodes = x.shape[0]

    mesh = plsc.VectorSubcoreMesh(core_axis_name="c", subcore_axis_name="s")

    @functools.partial(
        pl.kernel,
        out_type=jax.ShapeDtypeStruct((e_pad,), jnp.float32),
        mesh=mesh,
        scratch_types=[
            [pltpu.VMEM((C,), jnp.int32) for _ in range(3)],
            [pltpu.VMEM((C,), jnp.int32) for _ in range(3)],
            [pltpu.VMEM((C, D), jnp.float32) for _ in range(2)],
            [pltpu.VMEM((C, D), jnp.float32) for _ in range(2)],
            pltpu.VMEM((epw,), jnp.float32),
            pltpu.VMEM((L * 17,), jnp.float32),
            pltpu.VMEM_SHARED((n_nodes, D), jnp.float32),
            [pltpu.SemaphoreType.DMA for _ in range(3)],
            [pltpu.SemaphoreType.DMA for _ in range(3)],
            [pltpu.SemaphoreType.DMA for _ in range(2)],
            [pltpu.SemaphoreType.DMA for _ in range(2)],
        ],
        compiler_params=pltpu.CompilerParams(needs_layout_passes=False),
    )
    def k(x_hbm, src_hbm, dst_hbm, out_hbm,
          ids_s, ids_d, rows_s, rows_d, sc, tp, x_sh,
          sem_is, sem_id, sem_a, sem_b):
        sid = lax.axis_index("s")
        wid = sid * NC + lax.axis_index("c")
        base = wid * epw

        stripe = ((n_nodes // NS) // 8) * 8
        tail = n_nodes - NS * stripe
        s_off = sid * stripe
        pltpu.sync_copy(x_hbm.at[pl.ds(s_off, stripe)],
                        x_sh.at[pl.ds(s_off, stripe)])
        if tail:
            @pl.when(sid == 0)
            def _():
                pltpu.sync_copy(x_hbm.at[pl.ds(NS * stripe, tail)],
                                x_sh.at[pl.ds(NS * stripe, tail)])
        plsc.subcore_barrier()

        def load_ids(chunk, q):
            off = base + chunk * C
            pltpu.async_copy(src_hbm.at[pl.ds(off, C)], ids_s[q], sem_is[q])
            pltpu.async_copy(dst_hbm.at[pl.ds(off, C)], ids_d[q], sem_id[q])

        def wait_ids(q):
            pltpu.make_async_copy(src_hbm.at[pl.ds(0, C)], ids_s[q], sem_is[q]).wait()
            pltpu.make_async_copy(dst_hbm.at[pl.ds(0, C)], ids_d[q], sem_id[q]).wait()

        def fetch(q, rp):
            pltpu.async_copy(x_sh.at[ids_s[q]], rows_s[rp], sem_a[rp])
            pltpu.async_copy(x_sh.at[ids_d[q]], rows_d[rp], sem_b[rp])

        def wait_rows(rp):
            pltpu.make_async_copy(x_hbm.at[pl.ds(0, C)], rows_s[rp], sem_a[rp]).wait()
            pltpu.make_async_copy(x_hbm.at[pl.ds(0, C)], rows_d[rp], sem_b[rp]).wait()

        def compute(chunk, rp):
            rs, rd = rows_s[rp], rows_d[rp]

            def group_body(g, carry2):
                for j in range(L):
                    e = g * L + j
                    ps = []
                    for kk in range(D // L):
                        a = rs[e, pl.ds(kk * L, L)]
                        b = rd[e, pl.ds(kk * L, L)]
                        ps.append(a * b)
                    s0 = (ps[0] + ps[1]) + (ps[2] + ps[3])
                    s1 = (ps[4] + ps[5]) + (ps[6] + ps[7])
                    tp[pl.ds(j * 17, L)] = s0 + s1
                base17 = lax.iota(jnp.int32, L) * 17
                acc0 = jnp.zeros((L,), jnp.float32)
                acc1 = jnp.zeros((L,), jnp.float32)
                for kk in range(L // 2):
                    acc0 = acc0 + plsc.load_gather(tp, [base17 + (2 * kk)])
                    acc1 = acc1 + plsc.load_gather(tp, [base17 + (2 * kk + 1)])
                sc[pl.ds(pl.multiple_of(chunk * C + g * L, L), L)] = acc0 + acc1
                return carry2

            lax.fori_loop(0, groups, group_body, 0, unroll=False)

        load_ids(jnp.int32(0), 0)
        load_ids(jnp.int32(1), 1)
        wait_ids(0)
        fetch(0, 0)

        def six_body(it, carry):
            c0 = 6 * it
            for b in range(6):
                c = c0 + b
                nxt = jnp.where(c + 1 == nchunk, 0, c + 1)
                nxt2 = jnp.where(c + 2 >= nchunk, c + 2 - nchunk, c + 2)
                wait_ids((b + 1) % 3)
                fetch((b + 1) % 3, (b + 1) % 2)
                load_ids(nxt2, (b + 2) % 3)
                wait_rows(b % 2)
                compute(c, b % 2)
            return carry

        lax.fori_loop(0, nchunk // 6, six_body, 0, unroll=False)
        wait_rows(0)
        wait_ids(1)
        pltpu.sync_copy(sc, out_hbm.at[pl.ds(base, epw)])

    return k(x, src2, dst2)


def kernel(x, edge_index):
    e = edge_index.shape[1]
    quantum = NW * C * 6
    e_pad = ((e + quantum - 1) // quantum) * quantum
    src = edge_index[0]
    dst = edge_index[1]
    if e_pad != e:
        pad = jnp.zeros((e_pad - e,), jnp.int32)
        src = jnp.concatenate([src, pad])
        dst = jnp.concatenate([dst, pad])
    out = _scores_padded(x, src, dst, e_pad)
    return out[:e]
